# Initial kernel scaffold; baseline (speedup 1.0000x reference)
import jax
import jax.numpy as jnp
from jax.experimental import pallas as pl


def kernel(x, edge_index, edge_attr, batch, src_node_idx, dest_node_idx, W_emb, b_emb, W1, b1, W2, b2, gamma, beta, W_ff, b_ff, W_ff1, b_ff1, W_ff2, b_ff2):
    raise NotImplementedError("write your pallas kernel here")


# speedup vs baseline: 3.7413x; 3.7413x over previous
import functools
import math

import jax
import jax.numpy as jnp
from jax import lax
from jax.experimental import pallas as pl
from jax.experimental.pallas import tpu as pltpu
from jax.experimental.pallas import tpu_sc as plsc

N = 10000
E = 320000
F = 128
CLS = 10
NB = 16
NPG = N // NB
NS = 16
NP = 10240
K = 80
EPT = E // NS
CH2 = EPT // K
NBUF = 5
PH = 4
PR = 3072
DUMP = 128
ACCR = PR + DUMP
PW = [PR] * 3 + [NP - 3 * PR]
SHFT = 14
RZ = ACCR // NS
RD = NP // NS
ZB = 40
RB = 400
GRID = N // RB
BN_SCALE = 1.0 / math.sqrt(1.0 + 1e-5)

_mesh1 = plsc.VectorSubcoreMesh(core_axis_name="c", subcore_axis_name="s",
                                num_cores=1)
_mesh2 = plsc.VectorSubcoreMesh(core_axis_name="c", subcore_axis_name="s")


@functools.partial(
    pl.kernel,
    out_type=jax.ShapeDtypeStruct((NP, 16), jnp.float32),
    mesh=_mesh1,
    scratch_types=[
        pltpu.VMEM((CH2, K), jnp.int32),
        pltpu.VMEM((NBUF, K), jnp.int32),
        pltpu.VMEM((K, 16), jnp.float32),
        pltpu.VMEM((ZB, 16), jnp.float32),
        pltpu.VMEM_SHARED((NP, 16), jnp.float32),
        pltpu.SemaphoreType.DMA((NBUF,)),
    ],
)
def _deg_kernel(dst_hbm, out_hbm, dst_v, rg_v, ones_v, zb_v, acc_sh, sem):
    s = lax.axis_index("s")
    pltpu.sync_copy(dst_hbm.at[s], dst_v)

    def fill(i, carry):
        ones_v[i, pl.ds(0, 16)] = jnp.full((16,), 1.0, jnp.float32)
        return carry

    lax.fori_loop(0, K, fill, 0)

    def fillz(i, carry):
        zb_v[i, pl.ds(0, 16)] = jnp.zeros((16,), jnp.float32)
        return carry

    lax.fori_loop(0, ZB, fillz, 0)

    def zero(r, carry):
        pltpu.sync_copy(zb_v, acc_sh.at[pl.ds(s * RD + r * ZB, ZB)])
        return carry

    lax.fori_loop(0, RD // ZB, zero, 0)
    plsc.subcore_barrier()

    def outer(r, carry):
        for b in range(NBUF):
            j = r * NBUF + b
            for t in range(K // 16):
                rg_v[b, pl.ds(16 * t, 16)] = dst_v[j, pl.ds(16 * t, 16)]
            pltpu.async_copy(ones_v, acc_sh.at[rg_v.at[b]], sem.at[b],
                             add=True)
        for b in range(NBUF):
            pltpu.make_async_copy(ones_v, acc_sh.at[rg_v.at[b]],
                                  sem.at[b]).wait()
        return carry

    lax.fori_loop(0, CH2 // NBUF, outer, 0)
    plsc.subcore_barrier()
    pltpu.sync_copy(acc_sh.at[pl.ds(s * RD, RD)],
                    out_hbm.at[pl.ds(s * RD, RD)])


@functools.partial(
    pl.kernel,
    out_type=[jax.ShapeDtypeStruct((w, F), jnp.float32) for w in PW],
    mesh=_mesh1,
    scratch_types=[
        pltpu.VMEM((CH2, K), jnp.int32),
        pltpu.VMEM((NBUF, K), jnp.int32),
        pltpu.VMEM((NBUF, K), jnp.int32),
        pltpu.VMEM((NBUF, K, F), jnp.float32),
        pltpu.VMEM((ZB, F), jnp.float32),
        pltpu.VMEM_SHARED((ACCR, F), jnp.float32),
        pltpu.SemaphoreType.DMA((NBUF,)),
        pltpu.SemaphoreType.DMA((NBUF,)),
    ],
)
def _edge_phase(h_hbm, pk_hbm, *refs):
    outs = refs[:PH]
    pk_v, srg_v, rmp_v, rows_v, zb_v, acc_sh, gsem, ssem = refs[PH:]
    s = lax.axis_index("s")
    pltpu.sync_copy(pk_hbm.at[s], pk_v)

    def fillz(i, carry):
        for t in range(F // 16):
            zb_v[i, pl.ds(16 * t, 16)] = jnp.zeros((16,), jnp.float32)
        return carry

    lax.fori_loop(0, ZB, fillz, 0)

    for p in range(PH):
        lo = p * PR
        plsc.subcore_barrier()

        def zero(r, carry):
            pltpu.sync_copy(zb_v, acc_sh.at[pl.ds(s * RZ + r * ZB, ZB)])
            return carry

        lax.fori_loop(0, RZ // ZB, zero, 0)
        plsc.subcore_barrier()

        def decode(b, j):
            for t in range(K // 16):
                p16 = pk_v[j, pl.ds(16 * t, 16)]
                s16 = p16 & ((1 << SHFT) - 1)
                d16 = lax.shift_right_logical(p16, SHFT)
                ok = (d16 >= lo) & (d16 < lo + PR)
                srg_v[b, pl.ds(16 * t, 16)] = s16
                rmp_v[b, pl.ds(16 * t, 16)] = jnp.where(
                    ok, d16 - (lo - DUMP), 0)

        for b in range(NBUF):
            decode(b, b)
            pltpu.async_copy(h_hbm.at[srg_v.at[b]], rows_v.at[b], gsem.at[b])

        def outer(r, carry):
            for b in range(NBUF):
                pltpu.make_async_copy(h_hbm.at[srg_v.at[b]], rows_v.at[b],
                                      gsem.at[b]).wait()
                pltpu.async_copy(rows_v.at[b], acc_sh.at[rmp_v.at[b]],
                                 ssem.at[b], add=True)
            for b in range(NBUF):
                pltpu.make_async_copy(rows_v.at[b], acc_sh.at[rmp_v.at[b]],
                                      ssem.at[b]).wait()

                @pl.when(r < CH2 // NBUF - 1)
                def _():
                    jn = (r + 1) * NBUF + b
                    decode(b, jn)
                    pltpu.async_copy(h_hbm.at[srg_v.at[b]], rows_v.at[b],
                                     gsem.at[b])
            return carry

        lax.fori_loop(0, CH2 // NBUF, outer, 0)
        plsc.subcore_barrier()
        rw = PW[p] // NS
        pltpu.sync_copy(acc_sh.at[pl.ds(DUMP + s * rw, rw)],
                        outs[p].at[pl.ds(s * rw, rw)])


def _dense_pre_body(x_ref, degp_ref, wemb_ref, bemb_ref, h0_ref, dv_ref):
    deg = degp_ref[:, 0:1] + 1.0
    dinv = lax.rsqrt(deg)
    h0_ref[...] = jnp.dot(x_ref[...], wemb_ref[...],
                          preferred_element_type=jnp.float32) + bemb_ref[...]
    dv_ref[...] = jnp.broadcast_to(dinv, (RB, F))


_dense_pre = pl.pallas_call(
    _dense_pre_body,
    grid=(GRID,),
    in_specs=[
        pl.BlockSpec((RB, F), lambda i: (i, 0)),
        pl.BlockSpec((RB, 16), lambda i: (i, 0)),
        pl.BlockSpec((F, F), lambda i: (0, 0)),
        pl.BlockSpec((1, F), lambda i: (0, 0)),
    ],
    out_specs=[
        pl.BlockSpec((RB, F), lambda i: (i, 0)),
        pl.BlockSpec((RB, F), lambda i: (i, 0)),
    ],
    out_shape=[
        jax.ShapeDtypeStruct((N, F), jnp.float32),
        jax.ShapeDtypeStruct((N, F), jnp.float32),
    ],
)


def _dense_mm_body(h_ref, w_ref, dv_ref, hp_ref):
    hp_ref[...] = jnp.dot(h_ref[...], w_ref[...],
                          preferred_element_type=jnp.float32) * dv_ref[...]


_dense_mm = pl.pallas_call(
    _dense_mm_body,
    grid=(GRID,),
    in_specs=[
        pl.BlockSpec((RB, F), lambda i: (i, 0)),
        pl.BlockSpec((F, F), lambda i: (0, 0)),
        pl.BlockSpec((RB, F), lambda i: (i, 0)),
    ],
    out_specs=pl.BlockSpec((RB, F), lambda i: (i, 0)),
    out_shape=jax.ShapeDtypeStruct((N, F), jnp.float32),
)


def _dense_post_body(acc_ref, hp_ref, dv_ref, b_ref, out_ref, relu_ref):
    out = dv_ref[...] * (acc_ref[...] + hp_ref[...]) + b_ref[...]
    out_ref[...] = out
    relu_ref[...] = jnp.maximum(out, 0.0)


_dense_post = pl.pallas_call(
    _dense_post_body,
    grid=(GRID,),
    in_specs=[
        pl.BlockSpec((RB, F), lambda i: (i, 0)),
        pl.BlockSpec((RB, F), lambda i: (i, 0)),
        pl.BlockSpec((RB, F), lambda i: (i, 0)),
        pl.BlockSpec((1, F), lambda i: (0, 0)),
    ],
    out_specs=[
        pl.BlockSpec((RB, F), lambda i: (i, 0)),
        pl.BlockSpec((RB, F), lambda i: (i, 0)),
    ],
    out_shape=[
        jax.ShapeDtypeStruct((N, F), jnp.float32),
        jax.ShapeDtypeStruct((N, F), jnp.float32),
    ],
)


def _gsel_body(h2_ref, srcl_ref, destl_ref, wfs_ref, wfd_ref, gadd_ref):
    ids = lax.broadcasted_iota(jnp.int32, (NB, N), 1)
    asrc = (ids == srcl_ref[...]).astype(jnp.float32)
    adst = (ids == destl_ref[...]).astype(jnp.float32)
    gsrc = jnp.dot(asrc, h2_ref[...], preferred_element_type=jnp.float32)
    gdst = jnp.dot(adst, h2_ref[...], preferred_element_type=jnp.float32)
    gadd_ref[...] = (
        jnp.dot(gsrc, wfs_ref[...], preferred_element_type=jnp.float32)
        + jnp.dot(gdst, wfd_ref[...], preferred_element_type=jnp.float32))


_gsel = pl.pallas_call(
    _gsel_body,
    out_shape=jax.ShapeDtypeStruct((NB, F), jnp.float32),
)


def _head_body(h2_ref, gadd_ref, wf0_ref, bff_ref, gsc_ref, beta_ref,
               wff1_ref, bff1_ref, wff2_ref, bff2_ref, out_ref):
    i = pl.program_id(0)
    rows = i * RB + lax.broadcasted_iota(jnp.int32, (RB, NB), 0)
    seg = rows // NPG
    sel = (seg == lax.broadcasted_iota(jnp.int32, (RB, NB), 1))
    addon = jnp.dot(sel.astype(jnp.float32), gadd_ref[...],
                    preferred_element_type=jnp.float32)
    f = (jnp.dot(h2_ref[...], wf0_ref[...],
                 preferred_element_type=jnp.float32)
         + bff_ref[...] + addon)
    f = f * (gsc_ref[...] * BN_SCALE) + beta_ref[...]
    f = jnp.maximum(f, 0.0)
    f = jnp.maximum(
        jnp.dot(f, wff1_ref[...], preferred_element_type=jnp.float32)
        + bff1_ref[...], 0.0)
    o = jnp.dot(f, wff2_ref[...],
                preferred_element_type=jnp.float32) + bff2_ref[...]
    m = jnp.max(o, axis=1, keepdims=True)
    lse = jnp.log(jnp.sum(jnp.exp(o - m), axis=1, keepdims=True)) + m
    out_ref[...] = o - lse


_head = pl.pallas_call(
    _head_body,
    grid=(GRID,),
    in_specs=[
        pl.BlockSpec((RB, F), lambda i: (i, 0)),
        pl.BlockSpec((NB, F), lambda i: (0, 0)),
        pl.BlockSpec((F, F), lambda i: (0, 0)),
        pl.BlockSpec((1, F), lambda i: (0, 0)),
        pl.BlockSpec((1, F), lambda i: (0, 0)),
        pl.BlockSpec((1, F), lambda i: (0, 0)),
        pl.BlockSpec((F, F), lambda i: (0, 0)),
        pl.BlockSpec((1, F), lambda i: (0, 0)),
        pl.BlockSpec((F, CLS), lambda i: (0, 0)),
        pl.BlockSpec((1, CLS), lambda i: (0, 0)),
    ],
    out_specs=pl.BlockSpec((RB, CLS), lambda i: (i, 0)),
    out_shape=jax.ShapeDtypeStruct((N, CLS), jnp.float32),
)


def kernel(x, edge_index, edge_attr, batch, src_node_idx, dest_node_idx,
           W_emb, b_emb, W1, b1, W2, b2, gamma, beta,
           W_ff, b_ff, W_ff1, b_ff1, W_ff2, b_ff2):
    dstw = edge_index[1].reshape(NS, CH2, K)
    packed = ((edge_index[1] << SHFT) | edge_index[0]).reshape(NS, CH2, K)

    degp = _deg_kernel(dstw)
    h0, dv = _dense_pre(x, degp, W_emb, b_emb.reshape(1, F))

    wstack = jnp.stack([W1, W2])
    bstack = jnp.stack([b1.reshape(1, F), b2.reshape(1, F)])

    def conv_body(h, wb):
        w, b = wb
        hp = _dense_mm(h, w, dv)

        accs = _edge_phase(hp, packed)
        acc = jnp.concatenate(accs, axis=0)
        out, hrelu = _dense_post(acc, hp, dv, b)
        return hrelu, out

    _, outs = lax.scan(conv_body, h0, (wstack, bstack))
    h2 = outs[1]

    offs = jnp.arange(NB, dtype=jnp.int32) * NPG
    srcl = (src_node_idx.astype(jnp.int32) + offs).reshape(NB, 1)
    destl = (dest_node_idx.astype(jnp.int32) + offs).reshape(NB, 1)
    gadd = _gsel(h2, srcl, destl, W_ff[F:2 * F], W_ff[2 * F:])
    return _head(h2, gadd, W_ff[:F], b_ff.reshape(1, F),
                 gamma.reshape(1, F), beta.reshape(1, F),
                 W_ff1, b_ff1.reshape(1, F), W_ff2, b_ff2.reshape(1, CLS))


# speedup vs baseline: 6.6095x; 1.7667x over previous
import functools
import math

import jax
import jax.numpy as jnp
from jax import lax
from jax.experimental import pallas as pl
from jax.experimental.pallas import tpu as pltpu
from jax.experimental.pallas import tpu_sc as plsc

N = 10000
E = 320000
F = 128
CLS = 10
NB = 16
NPG = N // NB
NS = 16
NP = 10240
K = 80
EPT = E // NS
CH2 = EPT // K
NBUF = 5
PH = 4
PR = 3072
DUMP = 128
ACCR = PR + DUMP
PW = [PR] * 3 + [NP - 3 * PR]
SHFT = 14
RZ = ACCR // NS
RD = NP // NS
ZB = 40
RB = 400
GRID = N // RB
BN_SCALE = 1.0 / math.sqrt(1.0 + 1e-5)

_mesh1 = plsc.VectorSubcoreMesh(core_axis_name="c", subcore_axis_name="s",
                                num_cores=1)
_mesh2 = plsc.VectorSubcoreMesh(core_axis_name="c", subcore_axis_name="s")


@functools.partial(
    pl.kernel,
    out_type=jax.ShapeDtypeStruct((NP, 16), jnp.float32),
    mesh=_mesh1,
    scratch_types=[
        pltpu.VMEM((CH2, K), jnp.int32),
        pltpu.VMEM((NBUF, K), jnp.int32),
        pltpu.VMEM((K, 16), jnp.float32),
        pltpu.VMEM((ZB, 16), jnp.float32),
        pltpu.VMEM_SHARED((NP, 16), jnp.float32),
        pltpu.SemaphoreType.DMA((NBUF,)),
    ],
)
def _deg_kernel(dst_hbm, out_hbm, dst_v, rg_v, ones_v, zb_v, acc_sh, sem):
    s = lax.axis_index("s")
    pltpu.sync_copy(dst_hbm.at[s], dst_v)

    def fill(i, carry):
        ones_v[i, pl.ds(0, 16)] = jnp.full((16,), 1.0, jnp.float32)
        return carry

    lax.fori_loop(0, K, fill, 0)

    def fillz(i, carry):
        zb_v[i, pl.ds(0, 16)] = jnp.zeros((16,), jnp.float32)
        return carry

    lax.fori_loop(0, ZB, fillz, 0)

    def zero(r, carry):
        pltpu.sync_copy(zb_v, acc_sh.at[pl.ds(s * RD + r * ZB, ZB)])
        return carry

    lax.fori_loop(0, RD // ZB, zero, 0)
    plsc.subcore_barrier()

    def outer(r, carry):
        for b in range(NBUF):
            j = r * NBUF + b
            for t in range(K // 16):
                rg_v[b, pl.ds(16 * t, 16)] = dst_v[j, pl.ds(16 * t, 16)]
            pltpu.async_copy(ones_v, acc_sh.at[rg_v.at[b]], sem.at[b],
                             add=True)
        for b in range(NBUF):
            pltpu.make_async_copy(ones_v, acc_sh.at[rg_v.at[b]],
                                  sem.at[b]).wait()
        return carry

    lax.fori_loop(0, CH2 // NBUF, outer, 0)
    plsc.subcore_barrier()
    pltpu.sync_copy(acc_sh.at[pl.ds(s * RD, RD)],
                    out_hbm.at[pl.ds(s * RD, RD)])


@functools.partial(
    pl.kernel,
    out_type=[jax.ShapeDtypeStruct((w, F), jnp.float32) for w in PW],
    mesh=_mesh2,
    scratch_types=[
        pltpu.VMEM((CH2, K), jnp.int32),
        pltpu.VMEM((NBUF, K), jnp.int32),
        pltpu.VMEM((NBUF, K), jnp.int32),
        pltpu.VMEM((NBUF, K, F), jnp.float32),
        pltpu.VMEM((ZB, F), jnp.float32),
        pltpu.VMEM_SHARED((ACCR, F), jnp.float32),
        pltpu.SemaphoreType.DMA((NBUF,)),
        pltpu.SemaphoreType.DMA((NBUF,)),
    ],
)
def _edge_phase(h_hbm, pk_hbm, *refs):
    outs = refs[:PH]
    pk_v, srg_v, rmp_v, rows_v, zb_v, acc_sh, gsem, ssem = refs[PH:]
    c = lax.axis_index("c")
    s = lax.axis_index("s")
    pltpu.sync_copy(pk_hbm.at[s], pk_v)

    def fillz(i, carry):
        for t in range(F // 16):
            zb_v[i, pl.ds(16 * t, 16)] = jnp.zeros((16,), jnp.float32)
        return carry

    lax.fori_loop(0, ZB, fillz, 0)

    for p in range(PH // 2):
        lo = (2 * p + c) * PR
        plsc.subcore_barrier()

        def zero(r, carry):
            pltpu.sync_copy(zb_v, acc_sh.at[pl.ds(s * RZ + r * ZB, ZB)])
            return carry

        lax.fori_loop(0, RZ // ZB, zero, 0)
        plsc.subcore_barrier()

        def decode(b, j):
            for t in range(K // 16):
                p16 = pk_v[j, pl.ds(16 * t, 16)]
                s16 = p16 & ((1 << SHFT) - 1)
                d16 = lax.shift_right_logical(p16, SHFT)
                ok = (d16 >= lo) & (d16 < lo + PR)
                srg_v[b, pl.ds(16 * t, 16)] = s16
                rmp_v[b, pl.ds(16 * t, 16)] = jnp.where(
                    ok, d16 - (lo - DUMP), 0)

        for b in range(NBUF):
            decode(b, b)
            pltpu.async_copy(h_hbm.at[srg_v.at[b]], rows_v.at[b], gsem.at[b])

        def outer(r, carry):
            for b in range(NBUF):
                pltpu.make_async_copy(h_hbm.at[srg_v.at[b]], rows_v.at[b],
                                      gsem.at[b]).wait()
                pltpu.async_copy(rows_v.at[b], acc_sh.at[rmp_v.at[b]],
                                 ssem.at[b], add=True)
            for b in range(NBUF):
                pltpu.make_async_copy(rows_v.at[b], acc_sh.at[rmp_v.at[b]],
                                      ssem.at[b]).wait()

                @pl.when(r < CH2 // NBUF - 1)
                def _():
                    jn = (r + 1) * NBUF + b
                    decode(b, jn)
                    pltpu.async_copy(h_hbm.at[srg_v.at[b]], rows_v.at[b],
                                     gsem.at[b])
            return carry

        lax.fori_loop(0, CH2 // NBUF, outer, 0)
        plsc.subcore_barrier()
        for cc in range(2):
            w = 2 * p + cc
            rw = PW[w] // NS

            @pl.when(c == cc)
            def _():
                pltpu.sync_copy(acc_sh.at[pl.ds(DUMP + s * rw, rw)],
                                outs[w].at[pl.ds(s * rw, rw)])


def _dense_pre_body(x_ref, degp_ref, wemb_ref, bemb_ref, h0_ref, dv_ref):
    deg = degp_ref[:, 0:1] + 1.0
    dinv = lax.rsqrt(deg)
    h0_ref[...] = jnp.dot(x_ref[...], wemb_ref[...],
                          preferred_element_type=jnp.float32) + bemb_ref[...]
    dv_ref[...] = jnp.broadcast_to(dinv, (RB, F))


_dense_pre = pl.pallas_call(
    _dense_pre_body,
    grid=(GRID,),
    in_specs=[
        pl.BlockSpec((RB, F), lambda i: (i, 0)),
        pl.BlockSpec((RB, 16), lambda i: (i, 0)),
        pl.BlockSpec((F, F), lambda i: (0, 0)),
        pl.BlockSpec((1, F), lambda i: (0, 0)),
    ],
    out_specs=[
        pl.BlockSpec((RB, F), lambda i: (i, 0)),
        pl.BlockSpec((RB, F), lambda i: (i, 0)),
    ],
    out_shape=[
        jax.ShapeDtypeStruct((N, F), jnp.float32),
        jax.ShapeDtypeStruct((N, F), jnp.float32),
    ],
)


def _dense_mm_body(h_ref, w_ref, dv_ref, hp_ref):
    hp_ref[...] = jnp.dot(h_ref[...], w_ref[...],
                          preferred_element_type=jnp.float32) * dv_ref[...]


_dense_mm = pl.pallas_call(
    _dense_mm_body,
    grid=(GRID,),
    in_specs=[
        pl.BlockSpec((RB, F), lambda i: (i, 0)),
        pl.BlockSpec((F, F), lambda i: (0, 0)),
        pl.BlockSpec((RB, F), lambda i: (i, 0)),
    ],
    out_specs=pl.BlockSpec((RB, F), lambda i: (i, 0)),
    out_shape=jax.ShapeDtypeStruct((N, F), jnp.float32),
)


def _dense_post_body(acc_ref, hp_ref, dv_ref, b_ref, out_ref, relu_ref):
    out = dv_ref[...] * (acc_ref[...] + hp_ref[...]) + b_ref[...]
    out_ref[...] = out
    relu_ref[...] = jnp.maximum(out, 0.0)


_dense_post = pl.pallas_call(
    _dense_post_body,
    grid=(GRID,),
    in_specs=[
        pl.BlockSpec((RB, F), lambda i: (i, 0)),
        pl.BlockSpec((RB, F), lambda i: (i, 0)),
        pl.BlockSpec((RB, F), lambda i: (i, 0)),
        pl.BlockSpec((1, F), lambda i: (0, 0)),
    ],
    out_specs=[
        pl.BlockSpec((RB, F), lambda i: (i, 0)),
        pl.BlockSpec((RB, F), lambda i: (i, 0)),
    ],
    out_shape=[
        jax.ShapeDtypeStruct((N, F), jnp.float32),
        jax.ShapeDtypeStruct((N, F), jnp.float32),
    ],
)


def _gsel_body(h2_ref, srcl_ref, destl_ref, wfs_ref, wfd_ref, gadd_ref):
    ids = lax.broadcasted_iota(jnp.int32, (NB, N), 1)
    asrc = (ids == srcl_ref[...]).astype(jnp.float32)
    adst = (ids == destl_ref[...]).astype(jnp.float32)
    gsrc = jnp.dot(asrc, h2_ref[...], preferred_element_type=jnp.float32)
    gdst = jnp.dot(adst, h2_ref[...], preferred_element_type=jnp.float32)
    gadd_ref[...] = (
        jnp.dot(gsrc, wfs_ref[...], preferred_element_type=jnp.float32)
        + jnp.dot(gdst, wfd_ref[...], preferred_element_type=jnp.float32))


_gsel = pl.pallas_call(
    _gsel_body,
    out_shape=jax.ShapeDtypeStruct((NB, F), jnp.float32),
)


def _head_body(h2_ref, gadd_ref, wf0_ref, bff_ref, gsc_ref, beta_ref,
               wff1_ref, bff1_ref, wff2_ref, bff2_ref, out_ref):
    i = pl.program_id(0)
    rows = i * RB + lax.broadcasted_iota(jnp.int32, (RB, NB), 0)
    seg = rows // NPG
    sel = (seg == lax.broadcasted_iota(jnp.int32, (RB, NB), 1))
    addon = jnp.dot(sel.astype(jnp.float32), gadd_ref[...],
                    preferred_element_type=jnp.float32)
    f = (jnp.dot(h2_ref[...], wf0_ref[...],
                 preferred_element_type=jnp.float32)
         + bff_ref[...] + addon)
    f = f * (gsc_ref[...] * BN_SCALE) + beta_ref[...]
    f = jnp.maximum(f, 0.0)
    f = jnp.maximum(
        jnp.dot(f, wff1_ref[...], preferred_element_type=jnp.float32)
        + bff1_ref[...], 0.0)
    o = jnp.dot(f, wff2_ref[...],
                preferred_element_type=jnp.float32) + bff2_ref[...]
    m = jnp.max(o, axis=1, keepdims=True)
    lse = jnp.log(jnp.sum(jnp.exp(o - m), axis=1, keepdims=True)) + m
    out_ref[...] = o - lse


_head = pl.pallas_call(
    _head_body,
    grid=(GRID,),
    in_specs=[
        pl.BlockSpec((RB, F), lambda i: (i, 0)),
        pl.BlockSpec((NB, F), lambda i: (0, 0)),
        pl.BlockSpec((F, F), lambda i: (0, 0)),
        pl.BlockSpec((1, F), lambda i: (0, 0)),
        pl.BlockSpec((1, F), lambda i: (0, 0)),
        pl.BlockSpec((1, F), lambda i: (0, 0)),
        pl.BlockSpec((F, F), lambda i: (0, 0)),
        pl.BlockSpec((1, F), lambda i: (0, 0)),
        pl.BlockSpec((F, CLS), lambda i: (0, 0)),
        pl.BlockSpec((1, CLS), lambda i: (0, 0)),
    ],
    out_specs=pl.BlockSpec((RB, CLS), lambda i: (i, 0)),
    out_shape=jax.ShapeDtypeStruct((N, CLS), jnp.float32),
)


def kernel(x, edge_index, edge_attr, batch, src_node_idx, dest_node_idx,
           W_emb, b_emb, W1, b1, W2, b2, gamma, beta,
           W_ff, b_ff, W_ff1, b_ff1, W_ff2, b_ff2):
    dstw = edge_index[1].reshape(NS, CH2, K)
    packed = ((edge_index[1] << SHFT) | edge_index[0]).reshape(NS, CH2, K)

    degp = _deg_kernel(dstw)
    h0, dv = _dense_pre(x, degp, W_emb, b_emb.reshape(1, F))

    wstack = jnp.stack([W1, W2])
    bstack = jnp.stack([b1.reshape(1, F), b2.reshape(1, F)])

    def conv_body(h, wb):
        w, b = wb
        hp = _dense_mm(h, w, dv)

        accs = _edge_phase(hp, packed)
        acc = jnp.concatenate(accs, axis=0)
        out, hrelu = _dense_post(acc, hp, dv, b)
        return hrelu, out

    _, outs = lax.scan(conv_body, h0, (wstack, bstack))
    h2 = outs[1]

    offs = jnp.arange(NB, dtype=jnp.int32) * NPG
    srcl = (src_node_idx.astype(jnp.int32) + offs).reshape(NB, 1)
    destl = (dest_node_idx.astype(jnp.int32) + offs).reshape(NB, 1)
    gadd = _gsel(h2, srcl, destl, W_ff[F:2 * F], W_ff[2 * F:])
    return _head(h2, gadd, W_ff[:F], b_ff.reshape(1, F),
                 gamma.reshape(1, F), beta.reshape(1, F),
                 W_ff1, b_ff1.reshape(1, F), W_ff2, b_ff2.reshape(1, CLS))
